# baseline (speedup 1.0000x reference)
import jax
import jax.numpy as jnp
from jax.experimental import pallas as pl
from jax.experimental.pallas import tpu as pltpu


def _pool_stats_kernel(x_ref, xp_ref, mx_ref, s_ref):
    b = pl.program_id(0)
    xv = x_ref[0]
    nb, t = xv.shape
    xp = jnp.sum(xv.reshape(nb, t // 4, 4), axis=-1) * 0.25
    xp_ref[0] = xp
    mx_ref[0] = jnp.mean(xp, axis=-1, keepdims=True)
    prod = jax.lax.dot_general(xp, xp, (((1,), (1,)), ((), ())),
                               preferred_element_type=jnp.float32)

    @pl.when(b == 0)
    def _():
        s_ref[...] = prod

    @pl.when(b != 0)
    def _():
        s_ref[...] += prod


def _gate_fold_kernel(n_experts, n_count,
                      mx_ref, s_ref, gw_ref, gb_ref, cw_ref, cb_ref,
                      gam_ref, bet_ref,
                      wf_ref, bf_ref, eidx_ref, ew_ref):
    mx = mx_ref[...]
    bsz, _ = mx.shape
    n_logits = gw_ref.shape[0]
    logits = jax.lax.dot_general(mx, gw_ref[...], (((1,), (1,)), ((), ())),
                                 preferred_element_type=jnp.float32)
    logits = logits + gb_ref[...]
    z = logits - jnp.max(logits, axis=-1, keepdims=True)
    ez = jnp.exp(z)
    sm = ez / jnp.sum(ez, axis=-1, keepdims=True)
    cols = jax.lax.broadcasted_iota(jnp.int32, sm.shape, 1)
    v1 = jnp.max(sm, axis=-1, keepdims=True)
    a1 = jnp.min(jnp.where(sm == v1, cols, n_logits), axis=-1, keepdims=True)
    sm2 = jnp.where(cols == a1, -1.0, sm)
    v2 = jnp.max(sm2, axis=-1, keepdims=True)
    a2 = jnp.min(jnp.where(sm2 == v2, cols, n_logits), axis=-1, keepdims=True)
    den = v1 + v2
    w1 = jnp.where(a1 < n_experts, v1 / den, 0.0)
    w2 = jnp.where(a2 < n_experts, v2 / den, 0.0)
    e1 = jnp.minimum(a1, n_experts - 1)
    e2 = jnp.minimum(a2, n_experts - 1)
    eidx_ref[...] = jnp.concatenate([e1, e2], axis=1)
    ew_ref[...] = jnp.concatenate([w1, w2], axis=1)

    mean_all = jnp.mean(mx, axis=0, keepdims=True)
    outer = jax.lax.dot_general(mean_all, mean_all, (((0,), (0,)), ((), ())),
                                preferred_element_type=jnp.float32)
    cov = s_ref[...] * (1.0 / n_count) - outer
    cw = cw_ref[...]
    ws = jax.lax.dot_general(cw, cov, (((1,), (0,)), ((), ())),
                             preferred_element_type=jnp.float32)
    var = jnp.sum(ws * cw, axis=-1, keepdims=True)
    mu_x = jax.lax.dot_general(cw, mean_all, (((1,), (1,)), ((), ())),
                               preferred_element_type=jnp.float32)
    inv = gam_ref[...] * jax.lax.rsqrt(var + 1e-5)
    wf_ref[...] = cw * inv
    bf_ref[...] = -mu_x * inv + bet_ref[...]


def _moe_kernel(eidx_ref, ew_ref, wf_ref, bf_ref, xp_ref, out_ref):
    b = pl.program_id(0)
    k = pl.program_id(1)
    w = ew_ref[b, k]
    z = jax.lax.dot_general(wf_ref[0], xp_ref[0], (((1,), (0,)), ((), ())),
                            preferred_element_type=jnp.float32)
    y = jnp.maximum(z + bf_ref[0], 0.0) * w

    @pl.when(k == 0)
    def _():
        out_ref[0] = y

    @pl.when(k != 0)
    def _():
        out_ref[0] += y


def kernel(x, conv_w, conv_b, bn_gamma, bn_beta, gate_w, gate_b):
    B, NB, T = x.shape
    E, C, _ = conv_w.shape
    P = 4
    K = 2
    TP = T // P
    N = B * TP

    f32 = jnp.float32

    xp, mx3, s = pl.pallas_call(
        _pool_stats_kernel,
        grid=(B,),
        in_specs=[pl.BlockSpec((1, NB, T), lambda b: (b, 0, 0))],
        out_specs=[pl.BlockSpec((1, NB, TP), lambda b: (b, 0, 0)),
                   pl.BlockSpec((1, NB, 1), lambda b: (b, 0, 0)),
                   pl.BlockSpec((NB, NB), lambda b: (0, 0))],
        out_shape=[jax.ShapeDtypeStruct((B, NB, TP), f32),
                   jax.ShapeDtypeStruct((B, NB, 1), f32),
                   jax.ShapeDtypeStruct((NB, NB), f32)],
    )(x)
    mx = mx3.reshape(B, NB)

    import functools
    gate_fold = functools.partial(_gate_fold_kernel, E, N)
    wf_flat, bf_flat, eidx, ew = pl.pallas_call(
        gate_fold,
        out_shape=[jax.ShapeDtypeStruct((E * C, NB), f32),
                   jax.ShapeDtypeStruct((E * C, 1), f32),
                   jax.ShapeDtypeStruct((B, K), jnp.int32),
                   jax.ShapeDtypeStruct((B, K), f32)],
    )(mx, s, gate_w, gate_b.reshape(1, C),
      conv_w.reshape(E * C, NB), conv_b.reshape(E * C, 1),
      bn_gamma.reshape(E * C, 1), bn_beta.reshape(E * C, 1))

    wf = wf_flat.reshape(E, C, NB)
    bf = bf_flat.reshape(E, C, 1)

    out = pl.pallas_call(
        _moe_kernel,
        grid_spec=pltpu.PrefetchScalarGridSpec(
            num_scalar_prefetch=2,
            grid=(B, K),
            in_specs=[
                pl.BlockSpec((1, C, NB), lambda b, k, ei, w: (ei[b, k], 0, 0)),
                pl.BlockSpec((1, C, 1), lambda b, k, ei, w: (ei[b, k], 0, 0)),
                pl.BlockSpec((1, NB, TP), lambda b, k, ei, w: (b, 0, 0)),
            ],
            out_specs=pl.BlockSpec((1, C, TP), lambda b, k, ei, w: (b, 0, 0)),
        ),
        out_shape=jax.ShapeDtypeStruct((B, C, TP), f32),
    )(eidx, ew, wf, bf, xp)
    return out


# speedup vs baseline: 3.8125x; 3.8125x over previous
import jax
import jax.numpy as jnp
from jax.experimental import pallas as pl
from jax.experimental.pallas import tpu as pltpu


def _pool_kernel(x_ref, xp_ref):
    xv = x_ref[0]
    rows, lanes = xv.shape
    wlanes = lanes // 4
    ri = jax.lax.broadcasted_iota(jnp.int32, (lanes, wlanes), 0)
    ci = jax.lax.broadcasted_iota(jnp.int32, (lanes, wlanes), 1)
    pm = jnp.where(ri // 4 == ci, 0.25, 0.0).astype(jnp.float32)
    xp_ref[0] = jax.lax.dot_general(xv, pm, (((1,), (0,)), ((), ())),
                                    preferred_element_type=jnp.float32)


def _stats_kernel(xp_ref, mx_ref, s_ref):
    b = pl.program_id(0)
    xp = xp_ref[0]
    mx_ref[0] = jnp.mean(xp, axis=-1, keepdims=True)
    prod = jax.lax.dot_general(xp, xp, (((1,), (1,)), ((), ())),
                               preferred_element_type=jnp.float32)

    @pl.when(b == 0)
    def _():
        s_ref[...] = prod

    @pl.when(b != 0)
    def _():
        s_ref[...] += prod


def _gate_fold_kernel(n_experts, n_count,
                      mx_ref, s_ref, gw_ref, gb_ref, cw_ref, cb_ref,
                      gam_ref, bet_ref,
                      wf_ref, bf_ref, eidx_ref, ew_ref):
    mx = mx_ref[...]
    bsz, _ = mx.shape
    n_logits = gw_ref.shape[0]
    logits = jax.lax.dot_general(mx, gw_ref[...], (((1,), (1,)), ((), ())),
                                 preferred_element_type=jnp.float32)
    logits = logits + gb_ref[...]
    z = logits - jnp.max(logits, axis=-1, keepdims=True)
    ez = jnp.exp(z)
    sm = ez / jnp.sum(ez, axis=-1, keepdims=True)
    cols = jax.lax.broadcasted_iota(jnp.int32, sm.shape, 1)
    v1 = jnp.max(sm, axis=-1, keepdims=True)
    a1 = jnp.min(jnp.where(sm == v1, cols, n_logits), axis=-1, keepdims=True)
    sm2 = jnp.where(cols == a1, -1.0, sm)
    v2 = jnp.max(sm2, axis=-1, keepdims=True)
    a2 = jnp.min(jnp.where(sm2 == v2, cols, n_logits), axis=-1, keepdims=True)
    den = v1 + v2
    w1 = jnp.where(a1 < n_experts, v1 / den, 0.0)
    w2 = jnp.where(a2 < n_experts, v2 / den, 0.0)
    e1 = jnp.minimum(a1, n_experts - 1)
    e2 = jnp.minimum(a2, n_experts - 1)
    eidx_ref[...] = jnp.concatenate([e1, e2], axis=1)
    ew_ref[...] = jnp.concatenate([w1, w2], axis=1)

    mean_all = jnp.mean(mx, axis=0, keepdims=True)
    outer = jax.lax.dot_general(mean_all, mean_all, (((0,), (0,)), ((), ())),
                                preferred_element_type=jnp.float32)
    cov = s_ref[...] * (1.0 / n_count) - outer
    cw = cw_ref[...]
    ws = jax.lax.dot_general(cw, cov, (((1,), (0,)), ((), ())),
                             preferred_element_type=jnp.float32)
    var = jnp.sum(ws * cw, axis=-1, keepdims=True)
    mu_x = jax.lax.dot_general(cw, mean_all, (((1,), (1,)), ((), ())),
                               preferred_element_type=jnp.float32)
    inv = gam_ref[...] * jax.lax.rsqrt(var + 1e-5)
    wf_ref[...] = cw * inv
    bf_ref[...] = -mu_x * inv + bet_ref[...]


def _moe_kernel(eidx_ref, ew_ref, wf_ref, bf_ref, xp_ref, out_ref):
    b = pl.program_id(0)
    k = pl.program_id(1)
    w = ew_ref[b, k]
    z = jax.lax.dot_general(wf_ref[0], xp_ref[0], (((1,), (0,)), ((), ())),
                            preferred_element_type=jnp.float32)
    y = jnp.maximum(z + bf_ref[0], 0.0) * w

    @pl.when(k == 0)
    def _():
        out_ref[0] = y

    @pl.when(k != 0)
    def _():
        out_ref[0] += y


def kernel(x, conv_w, conv_b, bn_gamma, bn_beta, gate_w, gate_b):
    B, NB, T = x.shape
    E, C, _ = conv_w.shape
    P = 4
    K = 2
    TP = T // P
    N = B * TP

    f32 = jnp.float32

    L = 512
    R = NB * (T // L)
    xr = x.reshape(B, R, L)
    xp_r = pl.pallas_call(
        _pool_kernel,
        grid=(B,),
        in_specs=[pl.BlockSpec((1, R, L), lambda b: (b, 0, 0))],
        out_specs=pl.BlockSpec((1, R, L // 4), lambda b: (b, 0, 0)),
        out_shape=jax.ShapeDtypeStruct((B, R, L // 4), f32),
    )(xr)
    xp = xp_r.reshape(B, NB, TP)

    mx3, s = pl.pallas_call(
        _stats_kernel,
        grid=(B,),
        in_specs=[pl.BlockSpec((1, NB, TP), lambda b: (b, 0, 0))],
        out_specs=[pl.BlockSpec((1, NB, 1), lambda b: (b, 0, 0)),
                   pl.BlockSpec((NB, NB), lambda b: (0, 0))],
        out_shape=[jax.ShapeDtypeStruct((B, NB, 1), f32),
                   jax.ShapeDtypeStruct((NB, NB), f32)],
    )(xp)
    mx = mx3.reshape(B, NB)

    import functools
    gate_fold = functools.partial(_gate_fold_kernel, E, N)
    wf_flat, bf_flat, eidx, ew = pl.pallas_call(
        gate_fold,
        out_shape=[jax.ShapeDtypeStruct((E * C, NB), f32),
                   jax.ShapeDtypeStruct((E * C, 1), f32),
                   jax.ShapeDtypeStruct((B, K), jnp.int32),
                   jax.ShapeDtypeStruct((B, K), f32)],
    )(mx, s, gate_w, gate_b.reshape(1, C),
      conv_w.reshape(E * C, NB), conv_b.reshape(E * C, 1),
      bn_gamma.reshape(E * C, 1), bn_beta.reshape(E * C, 1))

    wf = wf_flat.reshape(E, C, NB)
    bf = bf_flat.reshape(E, C, 1)

    out = pl.pallas_call(
        _moe_kernel,
        grid_spec=pltpu.PrefetchScalarGridSpec(
            num_scalar_prefetch=2,
            grid=(B, K),
            in_specs=[
                pl.BlockSpec((1, C, NB), lambda b, k, ei, w: (ei[b, k], 0, 0)),
                pl.BlockSpec((1, C, 1), lambda b, k, ei, w: (ei[b, k], 0, 0)),
                pl.BlockSpec((1, NB, TP), lambda b, k, ei, w: (b, 0, 0)),
            ],
            out_specs=pl.BlockSpec((1, C, TP), lambda b, k, ei, w: (b, 0, 0)),
        ),
        out_shape=jax.ShapeDtypeStruct((B, C, TP), f32),
    )(eidx, ew, wf, bf, xp)
    return out


# speedup vs baseline: 7.7479x; 2.0322x over previous
import functools

import jax
import jax.numpy as jnp
from jax.experimental import pallas as pl
from jax.experimental.pallas import tpu as pltpu


def _pool_stats_kernel(x_ref, xp_ref, mx_ref, s_ref):
    b = pl.program_id(0)
    xv = x_ref[0]
    nb, t = xv.shape
    chunk = 512
    w = chunk // 4
    ri = jax.lax.broadcasted_iota(jnp.int32, (chunk, w), 0)
    ci = jax.lax.broadcasted_iota(jnp.int32, (chunk, w), 1)
    pm = jnp.where(ri // 4 == ci, 0.25, 0.0).astype(jnp.float32)
    parts = []
    for j in range(t // chunk):
        xc = xv[:, j * chunk:(j + 1) * chunk]
        parts.append(jax.lax.dot_general(
            xc, pm, (((1,), (0,)), ((), ())),
            preferred_element_type=jnp.float32))
    xp = jnp.concatenate(parts, axis=1)
    xp_ref[0] = xp
    mx_ref[0] = jnp.mean(xp, axis=-1, keepdims=True)
    prod = jax.lax.dot_general(xp, xp, (((1,), (1,)), ((), ())),
                               preferred_element_type=jnp.float32)

    @pl.when(b == 0)
    def _():
        s_ref[...] = prod

    @pl.when(b != 0)
    def _():
        s_ref[...] += prod


def _gate_fold_kernel(n_experts, n_count,
                      mx_ref, s_ref, gw_ref, gb_ref, cw_ref,
                      gam_ref, bet_ref,
                      wf_ref, bf_ref, eidx_ref, ew_ref):
    mx = mx_ref[...]
    n_logits = gw_ref.shape[0]
    logits = jax.lax.dot_general(mx, gw_ref[...], (((1,), (1,)), ((), ())),
                                 preferred_element_type=jnp.float32)
    logits = logits + gb_ref[...]
    z = logits - jnp.max(logits, axis=-1, keepdims=True)
    ez = jnp.exp(z)
    sm = ez / jnp.sum(ez, axis=-1, keepdims=True)
    cols = jax.lax.broadcasted_iota(jnp.int32, sm.shape, 1)
    v1 = jnp.max(sm, axis=-1, keepdims=True)
    a1 = jnp.min(jnp.where(sm == v1, cols, n_logits), axis=-1, keepdims=True)
    sm2 = jnp.where(cols == a1, -1.0, sm)
    v2 = jnp.max(sm2, axis=-1, keepdims=True)
    a2 = jnp.min(jnp.where(sm2 == v2, cols, n_logits), axis=-1, keepdims=True)
    den = v1 + v2
    w1 = jnp.where(a1 < n_experts, v1 / den, 0.0)
    w2 = jnp.where(a2 < n_experts, v2 / den, 0.0)
    e1 = jnp.minimum(a1, n_experts - 1)
    e2 = jnp.minimum(a2, n_experts - 1)
    eidx_ref[...] = jnp.concatenate([e1, e2], axis=1)
    ew_ref[...] = jnp.concatenate([w1, w2], axis=1)

    mean_all = jnp.mean(mx, axis=0, keepdims=True)
    outer = jax.lax.dot_general(mean_all, mean_all, (((0,), (0,)), ((), ())),
                                preferred_element_type=jnp.float32)
    cov = s_ref[...] * (1.0 / n_count) - outer
    cw = cw_ref[...]
    ws = jax.lax.dot_general(cw, cov, (((1,), (0,)), ((), ())),
                             preferred_element_type=jnp.float32)
    var = jnp.sum(ws * cw, axis=-1, keepdims=True)
    mu_x = jax.lax.dot_general(cw, mean_all, (((1,), (1,)), ((), ())),
                               preferred_element_type=jnp.float32)
    inv = gam_ref[...] * jax.lax.rsqrt(var + 1e-5)
    wf_ref[...] = cw * inv
    bf_ref[...] = -mu_x * inv + bet_ref[...]


def _moe_kernel(eidx_ref, ew_ref, wfa_ref, wfb_ref, bfa_ref, bfb_ref,
                xp_ref, out_ref):
    b = pl.program_id(0)
    w0 = ew_ref[b, 0]
    w1 = ew_ref[b, 1]
    wcat = jnp.concatenate([wfa_ref[0], wfb_ref[0]], axis=0)
    z = jax.lax.dot_general(wcat, xp_ref[0], (((1,), (0,)), ((), ())),
                            preferred_element_type=jnp.float32)
    c = wfa_ref.shape[1]
    y0 = jnp.maximum(z[:c] + bfa_ref[0], 0.0)
    y1 = jnp.maximum(z[c:] + bfb_ref[0], 0.0)
    out_ref[0] = y0 * w0 + y1 * w1


def kernel(x, conv_w, conv_b, bn_gamma, bn_beta, gate_w, gate_b):
    B, NB, T = x.shape
    E, C, _ = conv_w.shape
    P = 4
    K = 2
    TP = T // P
    N = B * TP
    f32 = jnp.float32

    xp, mx3, s = pl.pallas_call(
        _pool_stats_kernel,
        grid=(B,),
        in_specs=[pl.BlockSpec((1, NB, T), lambda b: (b, 0, 0))],
        out_specs=[pl.BlockSpec((1, NB, TP), lambda b: (b, 0, 0)),
                   pl.BlockSpec((1, NB, 1), lambda b: (b, 0, 0)),
                   pl.BlockSpec((NB, NB), lambda b: (0, 0))],
        out_shape=[jax.ShapeDtypeStruct((B, NB, TP), f32),
                   jax.ShapeDtypeStruct((B, NB, 1), f32),
                   jax.ShapeDtypeStruct((NB, NB), f32)],
    )(x)
    mx = mx3.reshape(B, NB)

    gate_fold = functools.partial(_gate_fold_kernel, E, N)
    wf_flat, bf_flat, eidx, ew = pl.pallas_call(
        gate_fold,
        out_shape=[jax.ShapeDtypeStruct((E * C, NB), f32),
                   jax.ShapeDtypeStruct((E * C, 1), f32),
                   jax.ShapeDtypeStruct((B, K), jnp.int32),
                   jax.ShapeDtypeStruct((B, K), f32)],
    )(mx, s, gate_w, gate_b.reshape(1, C),
      conv_w.reshape(E * C, NB),
      bn_gamma.reshape(E * C, 1), bn_beta.reshape(E * C, 1))

    wf = wf_flat.reshape(E, C, NB)
    bf = bf_flat.reshape(E, C, 1)

    out = pl.pallas_call(
        _moe_kernel,
        grid_spec=pltpu.PrefetchScalarGridSpec(
            num_scalar_prefetch=2,
            grid=(B,),
            in_specs=[
                pl.BlockSpec((1, C, NB), lambda b, ei, w: (ei[b, 0], 0, 0)),
                pl.BlockSpec((1, C, NB), lambda b, ei, w: (ei[b, 1], 0, 0)),
                pl.BlockSpec((1, C, 1), lambda b, ei, w: (ei[b, 0], 0, 0)),
                pl.BlockSpec((1, C, 1), lambda b, ei, w: (ei[b, 1], 0, 0)),
                pl.BlockSpec((1, NB, TP), lambda b, ei, w: (b, 0, 0)),
            ],
            out_specs=pl.BlockSpec((1, C, TP), lambda b, ei, w: (b, 0, 0)),
        ),
        out_shape=jax.ShapeDtypeStruct((B, C, TP), f32),
    )(eidx, ew, wf, wf, bf, bf, xp)
    return out


# speedup vs baseline: 10.4402x; 1.3475x over previous
import functools

import jax
import jax.numpy as jnp
from jax.experimental import pallas as pl
from jax.experimental.pallas import tpu as pltpu


def _pool_stats_kernel(x_ref, xp_ref, mx_ref, s_ref):
    b = pl.program_id(0)
    rows = x_ref.shape[0]
    nb, t = x_ref.shape[1], x_ref.shape[2]
    chunk = 512
    w = chunk // 4
    ri = jax.lax.broadcasted_iota(jnp.int32, (chunk, w), 0)
    ci = jax.lax.broadcasted_iota(jnp.int32, (chunk, w), 1)
    pm = jnp.where(ri // 4 == ci, 0.25, 0.0).astype(jnp.float32)
    prod = jnp.zeros((nb, nb), jnp.float32)
    for i in range(rows):
        xv = x_ref[i]
        parts = []
        for j in range(t // chunk):
            xc = xv[:, j * chunk:(j + 1) * chunk]
            parts.append(jax.lax.dot_general(
                xc, pm, (((1,), (0,)), ((), ())),
                preferred_element_type=jnp.float32))
        xp = jnp.concatenate(parts, axis=1)
        xp_ref[i] = xp.astype(jnp.bfloat16)
        mx_ref[i] = jnp.mean(xp, axis=-1, keepdims=True)
        prod = prod + jax.lax.dot_general(
            xp, xp, (((1,), (1,)), ((), ())),
            preferred_element_type=jnp.float32)

    @pl.when(b == 0)
    def _():
        s_ref[...] = prod

    @pl.when(b != 0)
    def _():
        s_ref[...] += prod


def _gate_fold_kernel(n_experts, n_count,
                      mx_ref, s_ref, gw_ref, gb_ref, cw_ref,
                      gam_ref, bet_ref,
                      wf_ref, bf_ref, eidx_ref, ew_ref):
    mx = mx_ref[...]
    n_logits = gw_ref.shape[0]
    logits = jax.lax.dot_general(mx, gw_ref[...], (((1,), (1,)), ((), ())),
                                 preferred_element_type=jnp.float32)
    logits = logits + gb_ref[...]
    z = logits - jnp.max(logits, axis=-1, keepdims=True)
    ez = jnp.exp(z)
    sm = ez / jnp.sum(ez, axis=-1, keepdims=True)
    cols = jax.lax.broadcasted_iota(jnp.int32, sm.shape, 1)
    v1 = jnp.max(sm, axis=-1, keepdims=True)
    a1 = jnp.min(jnp.where(sm == v1, cols, n_logits), axis=-1, keepdims=True)
    sm2 = jnp.where(cols == a1, -1.0, sm)
    v2 = jnp.max(sm2, axis=-1, keepdims=True)
    a2 = jnp.min(jnp.where(sm2 == v2, cols, n_logits), axis=-1, keepdims=True)
    den = v1 + v2
    w1 = jnp.where(a1 < n_experts, v1 / den, 0.0)
    w2 = jnp.where(a2 < n_experts, v2 / den, 0.0)
    e1 = jnp.minimum(a1, n_experts - 1)
    e2 = jnp.minimum(a2, n_experts - 1)
    eidx_ref[...] = jnp.concatenate([e1, e2], axis=1)
    ew_ref[...] = jnp.concatenate([w1, w2], axis=1)

    mean_all = jnp.mean(mx, axis=0, keepdims=True)
    outer = jax.lax.dot_general(mean_all, mean_all, (((0,), (0,)), ((), ())),
                                preferred_element_type=jnp.float32)
    cov = s_ref[...] * (1.0 / n_count) - outer
    cw = cw_ref[...]
    ws = jax.lax.dot_general(cw, cov, (((1,), (0,)), ((), ())),
                             preferred_element_type=jnp.float32)
    var = jnp.sum(ws * cw, axis=-1, keepdims=True)
    mu_x = jax.lax.dot_general(cw, mean_all, (((1,), (1,)), ((), ())),
                               preferred_element_type=jnp.float32)
    inv = gam_ref[...] * jax.lax.rsqrt(var + 1e-5)
    wf_ref[...] = (cw * inv).astype(jnp.bfloat16)
    bf_ref[...] = -mu_x * inv + bet_ref[...]


def _moe_kernel(eidx_ref, ew_ref, wfa0_ref, wfb0_ref, wfa1_ref, wfb1_ref,
                bfa0_ref, bfb0_ref, bfa1_ref, bfb1_ref, xp_ref, out_ref):
    b = pl.program_id(0)
    c = wfa0_ref.shape[1]

    def one_row(i, wfa_ref, wfb_ref, bfa_ref, bfb_ref):
        w0 = ew_ref[2 * b + i, 0]
        w1 = ew_ref[2 * b + i, 1]
        wcat = jnp.concatenate([wfa_ref[0], wfb_ref[0]], axis=0)
        z = jax.lax.dot_general(wcat, xp_ref[i], (((1,), (0,)), ((), ())),
                                preferred_element_type=jnp.float32)
        y0 = jnp.maximum(z[:c] + bfa_ref[0], 0.0)
        y1 = jnp.maximum(z[c:] + bfb_ref[0], 0.0)
        out_ref[i] = y0 * w0 + y1 * w1

    one_row(0, wfa0_ref, wfb0_ref, bfa0_ref, bfb0_ref)
    one_row(1, wfa1_ref, wfb1_ref, bfa1_ref, bfb1_ref)


def kernel(x, conv_w, conv_b, bn_gamma, bn_beta, gate_w, gate_b):
    B, NB, T = x.shape
    E, C, _ = conv_w.shape
    P = 4
    K = 2
    TP = T // P
    N = B * TP
    f32 = jnp.float32
    bf16 = jnp.bfloat16
    G = 2

    xp, mx3, s = pl.pallas_call(
        _pool_stats_kernel,
        grid=(B // G,),
        in_specs=[pl.BlockSpec((G, NB, T), lambda b: (b, 0, 0))],
        out_specs=[pl.BlockSpec((G, NB, TP), lambda b: (b, 0, 0)),
                   pl.BlockSpec((G, NB, 1), lambda b: (b, 0, 0)),
                   pl.BlockSpec((NB, NB), lambda b: (0, 0))],
        out_shape=[jax.ShapeDtypeStruct((B, NB, TP), bf16),
                   jax.ShapeDtypeStruct((B, NB, 1), f32),
                   jax.ShapeDtypeStruct((NB, NB), f32)],
    )(x)
    mx = mx3.reshape(B, NB)

    gate_fold = functools.partial(_gate_fold_kernel, E, N)
    wf_flat, bf_flat, eidx, ew = pl.pallas_call(
        gate_fold,
        out_shape=[jax.ShapeDtypeStruct((E * C, NB), bf16),
                   jax.ShapeDtypeStruct((E * C, 1), f32),
                   jax.ShapeDtypeStruct((B, K), jnp.int32),
                   jax.ShapeDtypeStruct((B, K), f32)],
    )(mx, s, gate_w, gate_b.reshape(1, C),
      conv_w.reshape(E * C, NB),
      bn_gamma.reshape(E * C, 1), bn_beta.reshape(E * C, 1))

    wf = wf_flat.reshape(E, C, NB)
    bf = bf_flat.reshape(E, C, 1)

    out = pl.pallas_call(
        _moe_kernel,
        grid_spec=pltpu.PrefetchScalarGridSpec(
            num_scalar_prefetch=2,
            grid=(B // G,),
            in_specs=[
                pl.BlockSpec((1, C, NB), lambda b, ei, w: (ei[2 * b, 0], 0, 0)),
                pl.BlockSpec((1, C, NB), lambda b, ei, w: (ei[2 * b, 1], 0, 0)),
                pl.BlockSpec((1, C, NB),
                             lambda b, ei, w: (ei[2 * b + 1, 0], 0, 0)),
                pl.BlockSpec((1, C, NB),
                             lambda b, ei, w: (ei[2 * b + 1, 1], 0, 0)),
                pl.BlockSpec((1, C, 1), lambda b, ei, w: (ei[2 * b, 0], 0, 0)),
                pl.BlockSpec((1, C, 1), lambda b, ei, w: (ei[2 * b, 1], 0, 0)),
                pl.BlockSpec((1, C, 1),
                             lambda b, ei, w: (ei[2 * b + 1, 0], 0, 0)),
                pl.BlockSpec((1, C, 1),
                             lambda b, ei, w: (ei[2 * b + 1, 1], 0, 0)),
                pl.BlockSpec((G, NB, TP), lambda b, ei, w: (b, 0, 0)),
            ],
            out_specs=pl.BlockSpec((G, C, TP), lambda b, ei, w: (b, 0, 0)),
        ),
        out_shape=jax.ShapeDtypeStruct((B, C, TP), f32),
    )(eidx, ew, wf, wf, wf, wf, bf, bf, bf, bf, xp)
    return out


# speedup vs baseline: 10.5213x; 1.0078x over previous
import functools

import jax
import jax.numpy as jnp
from jax.experimental import pallas as pl
from jax.experimental.pallas import tpu as pltpu


def _prep_kernel(n_experts, n_count,
                 x_ref, gw_ref, gb_ref, cw_ref, gam_ref, bet_ref,
                 xp_ref, wf_ref, bf_ref, eidx_ref, ew_ref,
                 mx_scr, s_scr):
    b = pl.program_id(0)
    nsteps = pl.num_programs(0)
    rows = x_ref.shape[0]
    nb, t = x_ref.shape[1], x_ref.shape[2]
    chunk = 512
    w = chunk // 4
    ri = jax.lax.broadcasted_iota(jnp.int32, (chunk, w), 0)
    ci = jax.lax.broadcasted_iota(jnp.int32, (chunk, w), 1)
    pm = jnp.where(ri // 4 == ci, 0.25, 0.0).astype(jnp.float32)
    bsz = mx_scr.shape[1]
    lane_b = jax.lax.broadcasted_iota(jnp.int32, (1, bsz), 1)
    prod = jnp.zeros((nb, nb), jnp.float32)
    mxadd = jnp.zeros((nb, bsz), jnp.float32)
    for i in range(rows):
        xv = x_ref[i]
        parts = []
        for j in range(t // chunk):
            xc = xv[:, j * chunk:(j + 1) * chunk]
            parts.append(jax.lax.dot_general(
                xc, pm, (((1,), (0,)), ((), ())),
                preferred_element_type=jnp.float32))
        xp = jnp.concatenate(parts, axis=1)
        xp_ref[i] = xp.astype(jnp.bfloat16)
        mean_i = jnp.mean(xp, axis=-1, keepdims=True)
        mxadd = mxadd + mean_i * (lane_b == rows * b + i)
        prod = prod + jax.lax.dot_general(
            xp, xp, (((1,), (1,)), ((), ())),
            preferred_element_type=jnp.float32)

    @pl.when(b == 0)
    def _():
        s_scr[...] = prod
        mx_scr[...] = mxadd

    @pl.when(b != 0)
    def _():
        s_scr[...] += prod
        mx_scr[...] += mxadd

    @pl.when(b == nsteps - 1)
    def _():
        mxt = mx_scr[...]
        n_logits = gw_ref.shape[0]
        logits = jax.lax.dot_general(
            gw_ref[...], mxt, (((1,), (0,)), ((), ())),
            preferred_element_type=jnp.float32) + gb_ref[...]
        z = logits - jnp.max(logits, axis=0, keepdims=True)
        ez = jnp.exp(z)
        sm = ez / jnp.sum(ez, axis=0, keepdims=True)
        rws = jax.lax.broadcasted_iota(jnp.int32, sm.shape, 0)
        v1 = jnp.max(sm, axis=0, keepdims=True)
        a1 = jnp.min(jnp.where(sm == v1, rws, n_logits), axis=0, keepdims=True)
        sm2 = jnp.where(rws == a1, -1.0, sm)
        v2 = jnp.max(sm2, axis=0, keepdims=True)
        a2 = jnp.min(jnp.where(sm2 == v2, rws, n_logits), axis=0,
                     keepdims=True)
        den = v1 + v2
        w1 = jnp.where(a1 < n_experts, v1 / den, 0.0)
        w2 = jnp.where(a2 < n_experts, v2 / den, 0.0)
        e1 = jnp.minimum(a1, n_experts - 1)
        e2 = jnp.minimum(a2, n_experts - 1)
        eidx_ref[...] = jnp.concatenate([e1, e2], axis=0)
        ew_ref[...] = jnp.concatenate([w1, w2], axis=0)

        mean_all = jnp.mean(mxt, axis=1, keepdims=True)
        outer = jax.lax.dot_general(
            mean_all, mean_all, (((1,), (1,)), ((), ())),
            preferred_element_type=jnp.float32)
        cov = s_scr[...] * (1.0 / n_count) - outer
        cw = cw_ref[...]
        ws = jax.lax.dot_general(cw, cov, (((1,), (0,)), ((), ())),
                                 preferred_element_type=jnp.float32)
        var = jnp.sum(ws * cw, axis=-1, keepdims=True)
        mu_x = jax.lax.dot_general(cw, mean_all, (((1,), (0,)), ((), ())),
                                   preferred_element_type=jnp.float32)
        inv = gam_ref[...] * jax.lax.rsqrt(var + 1e-5)
        wf_ref[...] = (cw * inv).astype(jnp.bfloat16)
        bf_ref[...] = -mu_x * inv + bet_ref[...]


def _moe_kernel(eidx_ref, ew_ref, wfa0_ref, wfb0_ref, wfa1_ref, wfb1_ref,
                bfa0_ref, bfb0_ref, bfa1_ref, bfb1_ref, xp_ref, out_ref):
    b = pl.program_id(0)
    c = wfa0_ref.shape[1]

    def one_row(i, wfa_ref, wfb_ref, bfa_ref, bfb_ref):
        w0 = ew_ref[0, 2 * b + i]
        w1 = ew_ref[1, 2 * b + i]
        wcat = jnp.concatenate([wfa_ref[0], wfb_ref[0]], axis=0)
        z = jax.lax.dot_general(wcat, xp_ref[i], (((1,), (0,)), ((), ())),
                                preferred_element_type=jnp.float32)
        y0 = jnp.maximum(z[:c] + bfa_ref[0], 0.0)
        y1 = jnp.maximum(z[c:] + bfb_ref[0], 0.0)
        out_ref[i] = y0 * w0 + y1 * w1

    one_row(0, wfa0_ref, wfb0_ref, bfa0_ref, bfb0_ref)
    one_row(1, wfa1_ref, wfb1_ref, bfa1_ref, bfb1_ref)


def kernel(x, conv_w, conv_b, bn_gamma, bn_beta, gate_w, gate_b):
    B, NB, T = x.shape
    E, C, _ = conv_w.shape
    P = 4
    K = 2
    TP = T // P
    N = B * TP
    f32 = jnp.float32
    bf16 = jnp.bfloat16
    G = 2

    prep = functools.partial(_prep_kernel, E, N)
    xp, wf_flat, bf_flat, eidx, ew = pl.pallas_call(
        prep,
        grid=(B // G,),
        in_specs=[pl.BlockSpec((G, NB, T), lambda b: (b, 0, 0)),
                  pl.BlockSpec((C, NB), lambda b: (0, 0)),
                  pl.BlockSpec((C, 1), lambda b: (0, 0)),
                  pl.BlockSpec((E * C, NB), lambda b: (0, 0)),
                  pl.BlockSpec((E * C, 1), lambda b: (0, 0)),
                  pl.BlockSpec((E * C, 1), lambda b: (0, 0))],
        out_specs=[pl.BlockSpec((G, NB, TP), lambda b: (b, 0, 0)),
                   pl.BlockSpec((E * C, NB), lambda b: (0, 0)),
                   pl.BlockSpec((E * C, 1), lambda b: (0, 0)),
                   pl.BlockSpec((K, B), lambda b: (0, 0)),
                   pl.BlockSpec((K, B), lambda b: (0, 0))],
        out_shape=[jax.ShapeDtypeStruct((B, NB, TP), bf16),
                   jax.ShapeDtypeStruct((E * C, NB), bf16),
                   jax.ShapeDtypeStruct((E * C, 1), f32),
                   jax.ShapeDtypeStruct((K, B), jnp.int32),
                   jax.ShapeDtypeStruct((K, B), f32)],
        scratch_shapes=[pltpu.VMEM((NB, B), f32),
                        pltpu.VMEM((NB, NB), f32)],
    )(x, gate_w, gate_b.reshape(C, 1),
      conv_w.reshape(E * C, NB),
      bn_gamma.reshape(E * C, 1), bn_beta.reshape(E * C, 1))

    wf = wf_flat.reshape(E, C, NB)
    bf = bf_flat.reshape(E, C, 1)

    out = pl.pallas_call(
        _moe_kernel,
        grid_spec=pltpu.PrefetchScalarGridSpec(
            num_scalar_prefetch=2,
            grid=(B // G,),
            in_specs=[
                pl.BlockSpec((1, C, NB), lambda b, ei, w: (ei[0, 2 * b], 0, 0)),
                pl.BlockSpec((1, C, NB), lambda b, ei, w: (ei[1, 2 * b], 0, 0)),
                pl.BlockSpec((1, C, NB),
                             lambda b, ei, w: (ei[0, 2 * b + 1], 0, 0)),
                pl.BlockSpec((1, C, NB),
                             lambda b, ei, w: (ei[1, 2 * b + 1], 0, 0)),
                pl.BlockSpec((1, C, 1), lambda b, ei, w: (ei[0, 2 * b], 0, 0)),
                pl.BlockSpec((1, C, 1), lambda b, ei, w: (ei[1, 2 * b], 0, 0)),
                pl.BlockSpec((1, C, 1),
                             lambda b, ei, w: (ei[0, 2 * b + 1], 0, 0)),
                pl.BlockSpec((1, C, 1),
                             lambda b, ei, w: (ei[1, 2 * b + 1], 0, 0)),
                pl.BlockSpec((G, NB, TP), lambda b, ei, w: (b, 0, 0)),
            ],
            out_specs=pl.BlockSpec((G, C, TP), lambda b, ei, w: (b, 0, 0)),
        ),
        out_shape=jax.ShapeDtypeStruct((B, C, TP), f32),
    )(eidx, ew, wf, wf, wf, wf, bf, bf, bf, bf, xp)
    return out


# speedup vs baseline: 11.8085x; 1.1223x over previous
import functools

import jax
import jax.numpy as jnp
from jax.experimental import pallas as pl
from jax.experimental.pallas import tpu as pltpu


def _fused_kernel(n_experts, n_count,
                  x_ref, gw_ref, gb_ref, cw_ref, gam_ref, bet_ref,
                  out_ref,
                  xp_scr, mx_scr, s_scr, wf_scr, bf_scr, eidx_scr, ew_scr):
    s = pl.program_id(0)
    nsteps = pl.num_programs(0)
    nprep = nsteps // 2
    rows = x_ref.shape[0]
    nb, t = x_ref.shape[1], x_ref.shape[2]
    bsz = mx_scr.shape[1]
    n_ch = out_ref.shape[1]

    @pl.when(s < nprep)
    def _prep():
        chunk = 512
        w = chunk // 4
        ri = jax.lax.broadcasted_iota(jnp.int32, (chunk, w), 0)
        ci = jax.lax.broadcasted_iota(jnp.int32, (chunk, w), 1)
        pm = jnp.where(ri // 4 == ci, 0.25, 0.0).astype(jnp.float32)
        lane_b = jax.lax.broadcasted_iota(jnp.int32, (1, bsz), 1)
        prod = jnp.zeros((nb, nb), jnp.float32)
        mxadd = jnp.zeros((nb, bsz), jnp.float32)
        for i in range(rows):
            xv = x_ref[i]
            parts = []
            for j in range(t // chunk):
                xc = xv[:, j * chunk:(j + 1) * chunk]
                parts.append(jax.lax.dot_general(
                    xc, pm, (((1,), (0,)), ((), ())),
                    preferred_element_type=jnp.float32))
            xp = jnp.concatenate(parts, axis=1)
            xp_scr[rows * s + i] = xp.astype(jnp.bfloat16)
            mean_i = jnp.mean(xp, axis=-1, keepdims=True)
            mxadd = mxadd + mean_i * (lane_b == rows * s + i)
            prod = prod + jax.lax.dot_general(
                xp, xp, (((1,), (1,)), ((), ())),
                preferred_element_type=jnp.float32)

        @pl.when(s == 0)
        def _():
            s_scr[...] = prod
            mx_scr[...] = mxadd

        @pl.when(s != 0)
        def _():
            s_scr[...] += prod
            mx_scr[...] += mxadd

        @pl.when(s == nprep - 1)
        def _gate_fold():
            mxt = mx_scr[...]
            n_logits = gw_ref.shape[0]
            logits = jax.lax.dot_general(
                gw_ref[...], mxt, (((1,), (0,)), ((), ())),
                preferred_element_type=jnp.float32) + gb_ref[...]
            z = logits - jnp.max(logits, axis=0, keepdims=True)
            ez = jnp.exp(z)
            sm = ez / jnp.sum(ez, axis=0, keepdims=True)
            rws = jax.lax.broadcasted_iota(jnp.int32, sm.shape, 0)
            v1 = jnp.max(sm, axis=0, keepdims=True)
            a1 = jnp.min(jnp.where(sm == v1, rws, n_logits), axis=0,
                         keepdims=True)
            sm2 = jnp.where(rws == a1, -1.0, sm)
            v2 = jnp.max(sm2, axis=0, keepdims=True)
            a2 = jnp.min(jnp.where(sm2 == v2, rws, n_logits), axis=0,
                         keepdims=True)
            den = v1 + v2
            w1 = jnp.where(a1 < n_experts, v1 / den, 0.0)
            w2 = jnp.where(a2 < n_experts, v2 / den, 0.0)
            e1 = jnp.minimum(a1, n_experts - 1)
            e2 = jnp.minimum(a2, n_experts - 1)
            eidx_scr[...] = jnp.concatenate([e1, e2], axis=0)
            ew_scr[...] = jnp.concatenate([w1, w2], axis=0)

            mean_all = jnp.mean(mxt, axis=1, keepdims=True)
            outer = jax.lax.dot_general(
                mean_all, mean_all, (((1,), (1,)), ((), ())),
                preferred_element_type=jnp.float32)
            cov = s_scr[...] * (1.0 / n_count) - outer
            cw = cw_ref[...]
            ws = jax.lax.dot_general(cw, cov, (((1,), (0,)), ((), ())),
                                     preferred_element_type=jnp.float32)
            var = jnp.sum(ws * cw, axis=-1, keepdims=True)
            mu_x = jax.lax.dot_general(cw, mean_all,
                                       (((1,), (0,)), ((), ())),
                                       preferred_element_type=jnp.float32)
            inv = gam_ref[...] * jax.lax.rsqrt(var + 1e-5)
            wff = (cw * inv).astype(jnp.bfloat16)
            bff = -mu_x * inv + bet_ref[...]
            for e in range(n_experts):
                wf_scr[e] = wff[e * n_ch:(e + 1) * n_ch]
                bf_scr[e] = bff[e * n_ch:(e + 1) * n_ch]

    @pl.when(s >= nprep)
    def _apply():
        pair = s - nprep
        lane_b = jax.lax.broadcasted_iota(jnp.int32, (1, bsz), 1)
        for i in range(rows):
            ridx = rows * pair + i
            onehot = (lane_b == ridx).astype(jnp.float32)
            w0 = jnp.sum(ew_scr[0:1, :] * onehot)
            w1 = jnp.sum(ew_scr[1:2, :] * onehot)
            e0 = jnp.sum(eidx_scr[0:1, :] * (lane_b == ridx))
            e1 = jnp.sum(eidx_scr[1:2, :] * (lane_b == ridx))
            wcat = jnp.concatenate([wf_scr[e0], wf_scr[e1]], axis=0)
            zz = jax.lax.dot_general(
                wcat, xp_scr[ridx], (((1,), (0,)), ((), ())),
                preferred_element_type=jnp.float32)
            y0 = jnp.maximum(zz[:n_ch] + bf_scr[e0], 0.0)
            y1 = jnp.maximum(zz[n_ch:] + bf_scr[e1], 0.0)
            out_ref[i] = y0 * w0 + y1 * w1


def kernel(x, conv_w, conv_b, bn_gamma, bn_beta, gate_w, gate_b):
    B, NB, T = x.shape
    E, C, _ = conv_w.shape
    P = 4
    TP = T // P
    N = B * TP
    f32 = jnp.float32
    bf16 = jnp.bfloat16
    G = 2
    NPREP = B // G

    fused = functools.partial(_fused_kernel, E, N)
    out = pl.pallas_call(
        fused,
        grid=(2 * NPREP,),
        in_specs=[
            pl.BlockSpec((G, NB, T), lambda s: (jnp.minimum(s, B // G - 1),
                                                0, 0)),
            pl.BlockSpec((C, NB), lambda s: (0, 0)),
            pl.BlockSpec((C, 1), lambda s: (0, 0)),
            pl.BlockSpec((E * C, NB), lambda s: (0, 0)),
            pl.BlockSpec((E * C, 1), lambda s: (0, 0)),
            pl.BlockSpec((E * C, 1), lambda s: (0, 0)),
        ],
        out_specs=pl.BlockSpec((G, C, TP),
                               lambda s: (jnp.maximum(s - B // G, 0), 0, 0)),
        out_shape=jax.ShapeDtypeStruct((B, C, TP), f32),
        scratch_shapes=[
            pltpu.VMEM((B, NB, TP), bf16),
            pltpu.VMEM((NB, B), f32),
            pltpu.VMEM((NB, NB), f32),
            pltpu.VMEM((E, C, NB), bf16),
            pltpu.VMEM((E, C, 1), f32),
            pltpu.VMEM((2, B), jnp.int32),
            pltpu.VMEM((2, B), f32),
        ],
    )(x, gate_w, gate_b.reshape(C, 1),
      conv_w.reshape(E * C, NB),
      bn_gamma.reshape(E * C, 1), bn_beta.reshape(E * C, 1))
    return out


# speedup vs baseline: 13.8021x; 1.1688x over previous
import functools

import jax
import jax.numpy as jnp
from jax.experimental import pallas as pl
from jax.experimental.pallas import tpu as pltpu


def _fused_kernel(n_experts, n_count,
                  x_ref, gw_ref, gb_ref, cw_ref, gam_ref, bet_ref,
                  out_ref,
                  xp_scr, mx_scr, s_scr, wf_scr, bf_scr, eidx_scr, ew_scr):
    s = pl.program_id(0)
    nsteps = pl.num_programs(0)
    nprep = nsteps // 2
    rows = x_ref.shape[0]
    nb, t = x_ref.shape[1], x_ref.shape[2]
    bsz = mx_scr.shape[1]
    n_ch = out_ref.shape[1]

    @pl.when(s < nprep)
    def _prep():
        chunk = 512
        w = chunk // 4
        ri = jax.lax.broadcasted_iota(jnp.int32, (chunk, w), 0)
        ci = jax.lax.broadcasted_iota(jnp.int32, (chunk, w), 1)
        pm = jnp.where(ri // 4 == ci, 0.25, 0.0).astype(jnp.float32)
        lane_b = jax.lax.broadcasted_iota(jnp.int32, (1, bsz), 1)
        prod = jnp.zeros((nb, nb), jnp.float32)
        mxadd = jnp.zeros((nb, bsz), jnp.float32)
        for i in range(rows):
            xv = x_ref[i]
            parts = []
            for j in range(t // chunk):
                xc = xv[:, j * chunk:(j + 1) * chunk]
                parts.append(jax.lax.dot_general(
                    xc, pm, (((1,), (0,)), ((), ())),
                    preferred_element_type=jnp.float32))
            xp = jnp.concatenate(parts, axis=1)
            xp_scr[rows * s + i] = xp.astype(jnp.bfloat16)
            mean_i = jnp.mean(xp, axis=-1, keepdims=True)
            mxadd = mxadd + mean_i * (lane_b == rows * s + i)
            prod = prod + jax.lax.dot_general(
                xp, xp, (((1,), (1,)), ((), ())),
                preferred_element_type=jnp.float32)

        @pl.when(s == 0)
        def _():
            s_scr[...] = prod
            mx_scr[...] = mxadd

        @pl.when(s != 0)
        def _():
            s_scr[...] += prod
            mx_scr[...] += mxadd

        @pl.when(s == nprep - 1)
        def _gate_fold():
            mxt = mx_scr[...]
            n_logits = gw_ref.shape[0]
            logits = jax.lax.dot_general(
                gw_ref[...], mxt, (((1,), (0,)), ((), ())),
                preferred_element_type=jnp.float32) + gb_ref[...]
            z = logits - jnp.max(logits, axis=0, keepdims=True)
            ez = jnp.exp(z)
            sm = ez / jnp.sum(ez, axis=0, keepdims=True)
            rws = jax.lax.broadcasted_iota(jnp.int32, sm.shape, 0)
            v1 = jnp.max(sm, axis=0, keepdims=True)
            a1 = jnp.min(jnp.where(sm == v1, rws, n_logits), axis=0,
                         keepdims=True)
            sm2 = jnp.where(rws == a1, -1.0, sm)
            v2 = jnp.max(sm2, axis=0, keepdims=True)
            a2 = jnp.min(jnp.where(sm2 == v2, rws, n_logits), axis=0,
                         keepdims=True)
            den = v1 + v2
            w1 = jnp.where(a1 < n_experts, v1 / den, 0.0)
            w2 = jnp.where(a2 < n_experts, v2 / den, 0.0)
            e1 = jnp.minimum(a1, n_experts - 1)
            e2 = jnp.minimum(a2, n_experts - 1)
            eidx_scr[...] = jnp.concatenate([e1, e2], axis=0)
            ew_scr[...] = jnp.concatenate([w1, w2], axis=0)

            mean_all = jnp.mean(mxt, axis=1, keepdims=True)
            outer = jax.lax.dot_general(
                mean_all, mean_all, (((1,), (1,)), ((), ())),
                preferred_element_type=jnp.float32)
            cov = s_scr[...] * (1.0 / n_count) - outer
            cw = cw_ref[...]
            ws = jax.lax.dot_general(cw, cov, (((1,), (0,)), ((), ())),
                                     preferred_element_type=jnp.float32)
            var = jnp.sum(ws * cw, axis=-1, keepdims=True)
            mu_x = jax.lax.dot_general(cw, mean_all,
                                       (((1,), (0,)), ((), ())),
                                       preferred_element_type=jnp.float32)
            inv = gam_ref[...] * jax.lax.rsqrt(var + 1e-5)
            wff = (cw * inv).astype(jnp.bfloat16)
            bff = -mu_x * inv + bet_ref[...]
            for e in range(n_experts):
                wf_scr[e] = wff[e * n_ch:(e + 1) * n_ch]
                bf_scr[e] = bff[e * n_ch:(e + 1) * n_ch]

    @pl.when(s >= nprep)
    def _apply():
        pair = s - nprep
        lane_b = jax.lax.broadcasted_iota(jnp.int32, (1, bsz), 1)
        for i in range(rows):
            ridx = rows * pair + i
            onehot = (lane_b == ridx).astype(jnp.float32)
            w0 = jnp.sum(ew_scr[0:1, :] * onehot)
            w1 = jnp.sum(ew_scr[1:2, :] * onehot)
            e0 = jnp.sum(eidx_scr[0:1, :] * (lane_b == ridx))
            e1 = jnp.sum(eidx_scr[1:2, :] * (lane_b == ridx))
            wcat = jnp.concatenate([wf_scr[e0], wf_scr[e1]], axis=0)
            zz = jax.lax.dot_general(
                wcat, xp_scr[ridx], (((1,), (0,)), ((), ())),
                preferred_element_type=jnp.float32)
            y0 = jnp.maximum(zz[:n_ch] + bf_scr[e0], 0.0)
            y1 = jnp.maximum(zz[n_ch:] + bf_scr[e1], 0.0)
            out_ref[i] = y0 * w0 + y1 * w1


def kernel(x, conv_w, conv_b, bn_gamma, bn_beta, gate_w, gate_b):
    B, NB, T = x.shape
    E, C, _ = conv_w.shape
    P = 4
    TP = T // P
    N = B * TP
    f32 = jnp.float32
    bf16 = jnp.bfloat16
    G = 4
    NPREP = B // G

    fused = functools.partial(_fused_kernel, E, N)
    out = pl.pallas_call(
        fused,
        grid=(2 * NPREP,),
        in_specs=[
            pl.BlockSpec((G, NB, T), lambda s: (jnp.minimum(s, B // G - 1),
                                                0, 0)),
            pl.BlockSpec((C, NB), lambda s: (0, 0)),
            pl.BlockSpec((C, 1), lambda s: (0, 0)),
            pl.BlockSpec((E * C, NB), lambda s: (0, 0)),
            pl.BlockSpec((E * C, 1), lambda s: (0, 0)),
            pl.BlockSpec((E * C, 1), lambda s: (0, 0)),
        ],
        out_specs=pl.BlockSpec((G, C, TP),
                               lambda s: (jnp.maximum(s - B // G, 0), 0, 0)),
        out_shape=jax.ShapeDtypeStruct((B, C, TP), f32),
        scratch_shapes=[
            pltpu.VMEM((B, NB, TP), bf16),
            pltpu.VMEM((NB, B), f32),
            pltpu.VMEM((NB, NB), f32),
            pltpu.VMEM((E, C, NB), bf16),
            pltpu.VMEM((E, C, 1), f32),
            pltpu.VMEM((2, B), jnp.int32),
            pltpu.VMEM((2, B), f32),
        ],
    )(x, gate_w, gate_b.reshape(C, 1),
      conv_w.reshape(E * C, NB),
      bn_gamma.reshape(E * C, 1), bn_beta.reshape(E * C, 1))
    return out


# speedup vs baseline: 14.5103x; 1.0513x over previous
import functools

import jax
import jax.numpy as jnp
from jax.experimental import pallas as pl
from jax.experimental.pallas import tpu as pltpu


def _fused_kernel(n_experts, n_count,
                  x_ref, gw_ref, gb_ref, cw_ref, gam_ref, bet_ref,
                  out_ref,
                  xp_scr, mx_scr, s_scr, wf_scr, bf_scr, eidx_scr, ew_scr):
    s = pl.program_id(0)
    nsteps = pl.num_programs(0)
    nprep = nsteps // 2
    rows = x_ref.shape[0]
    nb, t = x_ref.shape[1], x_ref.shape[2]
    bsz = mx_scr.shape[1]
    n_ch = out_ref.shape[1]

    @pl.when(s < nprep)
    def _prep():
        chunk = 512
        w = chunk // 4
        ri = jax.lax.broadcasted_iota(jnp.int32, (chunk, w), 0)
        ci = jax.lax.broadcasted_iota(jnp.int32, (chunk, w), 1)
        pm = jnp.where(ri // 4 == ci, 0.25, 0.0).astype(jnp.float32)
        lane_b = jax.lax.broadcasted_iota(jnp.int32, (1, bsz), 1)
        prod = jnp.zeros((nb, nb), jnp.float32)
        mxadd = jnp.zeros((nb, bsz), jnp.float32)
        for i in range(rows):
            xv = x_ref[i]
            parts = []
            for j in range(t // chunk):
                xc = xv[:, j * chunk:(j + 1) * chunk]
                parts.append(jax.lax.dot_general(
                    xc, pm, (((1,), (0,)), ((), ())),
                    preferred_element_type=jnp.float32))
            xp = jnp.concatenate(parts, axis=1)
            xp_scr[rows * s + i] = xp.astype(jnp.bfloat16)
            mean_i = jnp.mean(xp, axis=-1, keepdims=True)
            mxadd = mxadd + mean_i * (lane_b == rows * s + i)
            prod = prod + jax.lax.dot_general(
                xp, xp, (((1,), (1,)), ((), ())),
                preferred_element_type=jnp.float32)

        @pl.when(s == 0)
        def _():
            s_scr[...] = prod
            mx_scr[...] = mxadd

        @pl.when(s != 0)
        def _():
            s_scr[...] += prod
            mx_scr[...] += mxadd

        @pl.when(s == nprep - 1)
        def _gate_fold():
            mxt = mx_scr[...]
            n_logits = gw_ref.shape[0]
            logits = jax.lax.dot_general(
                gw_ref[...], mxt, (((1,), (0,)), ((), ())),
                preferred_element_type=jnp.float32) + gb_ref[...]
            z = logits - jnp.max(logits, axis=0, keepdims=True)
            ez = jnp.exp(z)
            sm = ez / jnp.sum(ez, axis=0, keepdims=True)
            rws = jax.lax.broadcasted_iota(jnp.int32, sm.shape, 0)
            v1 = jnp.max(sm, axis=0, keepdims=True)
            a1 = jnp.min(jnp.where(sm == v1, rws, n_logits), axis=0,
                         keepdims=True)
            sm2 = jnp.where(rws == a1, -1.0, sm)
            v2 = jnp.max(sm2, axis=0, keepdims=True)
            a2 = jnp.min(jnp.where(sm2 == v2, rws, n_logits), axis=0,
                         keepdims=True)
            den = v1 + v2
            w1 = jnp.where(a1 < n_experts, v1 / den, 0.0)
            w2 = jnp.where(a2 < n_experts, v2 / den, 0.0)
            e1 = jnp.minimum(a1, n_experts - 1)
            e2 = jnp.minimum(a2, n_experts - 1)
            eidx_scr[...] = jnp.concatenate([e1, e2], axis=0)
            ew_scr[...] = jnp.concatenate([w1, w2], axis=0)

            mean_all = jnp.mean(mxt, axis=1, keepdims=True)
            outer = jax.lax.dot_general(
                mean_all, mean_all, (((1,), (1,)), ((), ())),
                preferred_element_type=jnp.float32)
            cov = s_scr[...] * (1.0 / n_count) - outer
            cw = cw_ref[...]
            ws = jax.lax.dot_general(cw, cov, (((1,), (0,)), ((), ())),
                                     preferred_element_type=jnp.float32)
            var = jnp.sum(ws * cw, axis=-1, keepdims=True)
            mu_x = jax.lax.dot_general(cw, mean_all,
                                       (((1,), (0,)), ((), ())),
                                       preferred_element_type=jnp.float32)
            inv = gam_ref[...] * jax.lax.rsqrt(var + 1e-5)
            wff = (cw * inv).astype(jnp.bfloat16)
            bff = -mu_x * inv + bet_ref[...]
            for e in range(n_experts):
                wf_scr[e] = wff[e * n_ch:(e + 1) * n_ch]
                bf_scr[e] = bff[e * n_ch:(e + 1) * n_ch]

    @pl.when(s >= nprep)
    def _apply():
        pair = s - nprep
        lane_b = jax.lax.broadcasted_iota(jnp.int32, (1, bsz), 1)
        for i in range(rows):
            ridx = rows * pair + i
            onehot = (lane_b == ridx).astype(jnp.float32)
            w0 = jnp.sum(ew_scr[0:1, :] * onehot)
            w1 = jnp.sum(ew_scr[1:2, :] * onehot)
            e0 = jnp.sum(eidx_scr[0:1, :] * (lane_b == ridx))
            e1 = jnp.sum(eidx_scr[1:2, :] * (lane_b == ridx))
            wcat = jnp.concatenate([wf_scr[e0], wf_scr[e1]], axis=0)
            zz = jax.lax.dot_general(
                wcat, xp_scr[ridx], (((1,), (0,)), ((), ())),
                preferred_element_type=jnp.float32)
            y0 = jnp.maximum(zz[:n_ch] + bf_scr[e0], 0.0)
            y1 = jnp.maximum(zz[n_ch:] + bf_scr[e1], 0.0)
            out_ref[i] = y0 * w0 + y1 * w1


def kernel(x, conv_w, conv_b, bn_gamma, bn_beta, gate_w, gate_b):
    B, NB, T = x.shape
    E, C, _ = conv_w.shape
    P = 4
    TP = T // P
    N = B * TP
    f32 = jnp.float32
    bf16 = jnp.bfloat16
    G = 8
    NPREP = B // G

    fused = functools.partial(_fused_kernel, E, N)
    out = pl.pallas_call(
        fused,
        grid=(2 * NPREP,),
        in_specs=[
            pl.BlockSpec((G, NB, T), lambda s: (jnp.minimum(s, B // G - 1),
                                                0, 0)),
            pl.BlockSpec((C, NB), lambda s: (0, 0)),
            pl.BlockSpec((C, 1), lambda s: (0, 0)),
            pl.BlockSpec((E * C, NB), lambda s: (0, 0)),
            pl.BlockSpec((E * C, 1), lambda s: (0, 0)),
            pl.BlockSpec((E * C, 1), lambda s: (0, 0)),
        ],
        out_specs=pl.BlockSpec((G, C, TP),
                               lambda s: (jnp.maximum(s - B // G, 0), 0, 0)),
        out_shape=jax.ShapeDtypeStruct((B, C, TP), f32),
        scratch_shapes=[
            pltpu.VMEM((B, NB, TP), bf16),
            pltpu.VMEM((NB, B), f32),
            pltpu.VMEM((NB, NB), f32),
            pltpu.VMEM((E, C, NB), bf16),
            pltpu.VMEM((E, C, 1), f32),
            pltpu.VMEM((2, B), jnp.int32),
            pltpu.VMEM((2, B), f32),
        ],
    )(x, gate_w, gate_b.reshape(C, 1),
      conv_w.reshape(E * C, NB),
      bn_gamma.reshape(E * C, 1), bn_beta.reshape(E * C, 1))
    return out
